# baseline (speedup 1.0000x reference)
import functools

import jax
import jax.numpy as jnp
from jax import lax
from jax.experimental import pallas as pl
from jax.experimental.pallas import tpu as pltpu
from jax.experimental.pallas import tpu_sc as plsc

BATCH = 16384
NC, NS = 2, 16
NW = NC * NS
BPW = BATCH // NW
CHUNK = 128
NCHUNK = BPW // CHUNK
CHAR_D = 6
NCCHUNK = BPW * CHAR_D // CHUNK
LANES = 16
OUT_D = 28
BAND_D = 20
LOC_D, FACT_D, GOAL_D = 4, 6, 4


def _sc_encode(atype, achar6, aloc, afact, agoal,
               type_emb, char_flat, loc_emb8, fact_emb8, goal_emb8):
    mesh = plsc.VectorSubcoreMesh(core_axis_name="c", subcore_axis_name="s")

    @functools.partial(
        pl.kernel,
        mesh=mesh,
        compiler_params=pltpu.CompilerParams(
            use_tc_tiling_on_sc=False,
            needs_layout_passes=False),
        out_type=jax.ShapeDtypeStruct((BATCH, OUT_D), jnp.float32),
        scratch_types=[
            pltpu.VMEM((NCHUNK, CHUNK), jnp.int32),
            pltpu.VMEM((NCCHUNK, CHUNK), jnp.int32),
            pltpu.VMEM((NCHUNK, CHUNK), jnp.int32),
            pltpu.VMEM((NCHUNK, CHUNK), jnp.int32),
            pltpu.VMEM((NCHUNK, CHUNK), jnp.int32),
            pltpu.VMEM((BPW, 8), jnp.float32),
            pltpu.VMEM((BPW * CHAR_D,), jnp.float32),
            pltpu.VMEM((BPW, 8), jnp.float32),
            pltpu.VMEM((BPW, 8), jnp.float32),
            pltpu.VMEM((BPW, 8), jnp.float32),
            pltpu.VMEM((BPW, BAND_D), jnp.float32),
            pltpu.SemaphoreType.DMA,
            pltpu.SemaphoreType.DMA,
        ],
    )
    def k(atype_h, achar_h, aloc_h, afact_h, agoal_h,
          t_h, c_h, l_h, f_h, g_h, out_h,
          it_v, ic_v, il_v, if_v, ig_v,
          rt_v, rc_v, rl_v, rf_v, rg_v, band_v,
          sem_a, sem_b):
        wid = lax.axis_index("s") * NC + lax.axis_index("c")
        base = wid * BPW
        pltpu.sync_copy(achar_h.at[pl.ds(wid * NCCHUNK, NCCHUNK)], ic_v)
        for idx_h, idx_v in ((aloc_h, il_v), (afact_h, if_v),
                             (agoal_h, ig_v), (atype_h, it_v)):
            pltpu.sync_copy(idx_h.at[pl.ds(wid * NCHUNK, NCHUNK)], idx_v)

        band_gathers = []
        for j in range(NCCHUNK):
            band_gathers.append(pltpu.async_copy(
                c_h.at[ic_v.at[j]],
                rc_v.at[pl.ds(j * CHUNK, CHUNK)], sem_b))
        for tab_h, idx_v2, rows_v in ((l_h, il_v, rl_v),
                                      (f_h, if_v, rf_v),
                                      (g_h, ig_v, rg_v)):
            for j in range(NCHUNK):
                band_gathers.append(pltpu.async_copy(
                    tab_h.at[idx_v2.at[j]],
                    rows_v.at[pl.ds(j * CHUNK, CHUNK)], sem_b))
        tg = []
        for j in range(NCHUNK):
            tg.append(pltpu.async_copy(
                t_h.at[it_v.at[j]],
                rt_v.at[pl.ds(j * CHUNK, CHUNK)], sem_a))
        for cp in tg:
            cp.wait()
        out_t = pltpu.async_copy(
            rt_v, out_h.at[pl.ds(base, BPW), pl.ds(0, 8)], sem_a)
        for cp in band_gathers:
            cp.wait()

        iota = lax.iota(jnp.int32, LANES)

        cr, cc = [], []
        for p in range(3):
            e = iota + (p * LANES)
            cr.append(e // CHAR_D)
            cc.append(e % CHAR_D)

        def char_body(i, r):
            for p in range(3):
                v = rc_v[pl.ds((3 * i + p) * LANES, LANES)]
                plsc.store_scatter(band_v, [r + cr[p], cc[p]], v)
            return r + 8

        lax.fori_loop(0, BPW * CHAR_D // (3 * LANES), char_body,
                      iota * 0, unroll=2)

        cs4 = iota % 4
        r4 = iota // 4

        def loc_body(i, r):
            v = plsc.load_gather(rl_v, [r, cs4])
            plsc.store_scatter(band_v, [r, cs4 + CHAR_D], v)
            return r + 4

        lax.fori_loop(0, BPW * LOC_D // LANES, loc_body, r4, unroll=4)

        fr, fcs = [], []
        for p in range(3):
            e = iota + (p * LANES)
            fr.append(e // FACT_D)
            fcs.append(e % FACT_D)

        def fact_body(i, r):
            for p in range(3):
                rp = r + fr[p]
                v = plsc.load_gather(rf_v, [rp, fcs[p]])
                plsc.store_scatter(band_v, [rp, fcs[p] + CHAR_D + LOC_D], v)
            return r + 8

        lax.fori_loop(0, BPW * FACT_D // (3 * LANES), fact_body,
                      iota * 0, unroll=2)

        def goal_body(i, r):
            v = plsc.load_gather(rg_v, [r, cs4])
            plsc.store_scatter(band_v, [r, cs4 + (BAND_D - GOAL_D)], v)
            return r + 4

        lax.fori_loop(0, BPW * GOAL_D // LANES, goal_body, r4, unroll=4)

        pltpu.sync_copy(band_v, out_h.at[pl.ds(base, BPW), pl.ds(8, BAND_D)])
        out_t.wait()

    return k(atype, achar6, aloc, afact, agoal,
             type_emb, char_flat, loc_emb8, fact_emb8, goal_emb8)


def kernel(atype, achar, aloc, afact, agoal,
           type_emb, char_emb, loc_emb, fact_emb, goal_emb):
    def as_idx(a):
        return a.astype(jnp.int32).reshape(NW * NCHUNK, CHUNK)

    achar6 = (achar.astype(jnp.int32)[:, None] * CHAR_D
              + jnp.arange(CHAR_D, dtype=jnp.int32)[None, :]
              ).reshape(NW * NCCHUNK, CHUNK)

    def pad8(t):
        return jnp.pad(t, ((0, 0), (0, 8 - t.shape[1])))

    return _sc_encode(
        as_idx(atype), achar6, as_idx(aloc), as_idx(afact), as_idx(agoal),
        type_emb, char_emb.reshape(-1),
        pad8(loc_emb), pad8(fact_emb), pad8(goal_emb))


# speedup vs baseline: 1.0049x; 1.0049x over previous
import functools

import jax
import jax.numpy as jnp
from jax import lax
from jax.experimental import pallas as pl
from jax.experimental.pallas import tpu as pltpu
from jax.experimental.pallas import tpu_sc as plsc

BATCH = 16384
NC, NS = 2, 16
NW = NC * NS
BPW = BATCH // NW
CHUNK = 128
NCHUNK = BPW // CHUNK
CHAR_D = 6
NCCHUNK = BPW * CHAR_D // CHUNK
LANES = 16
OUT_D = 28
BAND_D = 20
LOC_D, FACT_D, GOAL_D = 4, 6, 4


def _sc_encode(atype, achar6, aloc, afact, agoal,
               type_emb, char_flat, loc_emb8, fact_emb8, goal_emb8):
    mesh = plsc.VectorSubcoreMesh(core_axis_name="c", subcore_axis_name="s")

    @functools.partial(
        pl.kernel,
        mesh=mesh,
        compiler_params=pltpu.CompilerParams(
            use_tc_tiling_on_sc=False,
            needs_layout_passes=False),
        out_type=jax.ShapeDtypeStruct((BATCH, OUT_D), jnp.float32),
        scratch_types=[
            pltpu.VMEM((NCHUNK, CHUNK), jnp.int32),
            pltpu.VMEM((NCCHUNK, CHUNK), jnp.int32),
            pltpu.VMEM((NCHUNK, CHUNK), jnp.int32),
            pltpu.VMEM((NCHUNK, CHUNK), jnp.int32),
            pltpu.VMEM((NCHUNK, CHUNK), jnp.int32),
            pltpu.VMEM((BPW, 8), jnp.float32),
            pltpu.VMEM((BPW * CHAR_D,), jnp.float32),
            pltpu.VMEM((BPW, 8), jnp.float32),
            pltpu.VMEM((BPW, 8), jnp.float32),
            pltpu.VMEM((BPW, 8), jnp.float32),
            pltpu.VMEM((BPW, BAND_D), jnp.float32),
            pltpu.SemaphoreType.DMA,
            pltpu.SemaphoreType.DMA,
        ],
    )
    def k(atype_h, achar_h, aloc_h, afact_h, agoal_h,
          t_h, c_h, l_h, f_h, g_h, out_h,
          it_v, ic_v, il_v, if_v, ig_v,
          rt_v, rc_v, rl_v, rf_v, rg_v, band_v,
          sem_a, sem_b):
        wid = lax.axis_index("s") * NC + lax.axis_index("c")
        base = wid * BPW
        pltpu.sync_copy(achar_h.at[pl.ds(wid * NCCHUNK, NCCHUNK)], ic_v)
        for idx_h, idx_v in ((aloc_h, il_v), (afact_h, if_v),
                             (agoal_h, ig_v), (atype_h, it_v)):
            pltpu.sync_copy(idx_h.at[pl.ds(wid * NCHUNK, NCHUNK)], idx_v)

        band_gathers = []
        for j in range(NCCHUNK):
            band_gathers.append(pltpu.async_copy(
                c_h.at[ic_v.at[j]],
                rc_v.at[pl.ds(j * CHUNK, CHUNK)], sem_b))
        for tab_h, idx_v2, rows_v in ((l_h, il_v, rl_v),
                                      (f_h, if_v, rf_v),
                                      (g_h, ig_v, rg_v)):
            for j in range(NCHUNK):
                band_gathers.append(pltpu.async_copy(
                    tab_h.at[idx_v2.at[j]],
                    rows_v.at[pl.ds(j * CHUNK, CHUNK)], sem_b))
        tg = []
        for j in range(NCHUNK):
            tg.append(pltpu.async_copy(
                t_h.at[it_v.at[j]],
                rt_v.at[pl.ds(j * CHUNK, CHUNK)], sem_a))
        with jax.named_scope("phase_tgwait"):
            for cp in tg:
                cp.wait()
        out_t = pltpu.async_copy(
            rt_v, out_h.at[pl.ds(base, BPW), pl.ds(0, 8)], sem_a)
        with jax.named_scope("phase_bandwait"):
            for cp in band_gathers:
                cp.wait()

        iota = lax.iota(jnp.int32, LANES)

        cr, cc = [], []
        for p in range(3):
            e = iota + (p * LANES)
            cr.append(e // CHAR_D)
            cc.append(e % CHAR_D)

        def char_body(i, r):
            for p in range(3):
                v = rc_v[pl.ds((3 * i + p) * LANES, LANES)]
                plsc.store_scatter(band_v, [r + cr[p], cc[p]], v)
            return r + 8

        with jax.named_scope("phase_cchar"):
            lax.fori_loop(0, BPW * CHAR_D // (3 * LANES), char_body,
                          iota * 0, unroll=2)

        cs4 = iota % 4
        r4 = iota // 4

        def loc_body(i, r):
            v = plsc.load_gather(rl_v, [r, cs4])
            plsc.store_scatter(band_v, [r, cs4 + CHAR_D], v)
            return r + 4

        with jax.named_scope("phase_cloc"):
            lax.fori_loop(0, BPW * LOC_D // LANES, loc_body, r4, unroll=4)

        fr, fcs = [], []
        for p in range(3):
            e = iota + (p * LANES)
            fr.append(e // FACT_D)
            fcs.append(e % FACT_D)

        def fact_body(i, r):
            for p in range(3):
                rp = r + fr[p]
                v = plsc.load_gather(rf_v, [rp, fcs[p]])
                plsc.store_scatter(band_v, [rp, fcs[p] + CHAR_D + LOC_D], v)
            return r + 8

        with jax.named_scope("phase_cfact"):
            lax.fori_loop(0, BPW * FACT_D // (3 * LANES), fact_body,
                          iota * 0, unroll=2)

        def goal_body(i, r):
            v = plsc.load_gather(rg_v, [r, cs4])
            plsc.store_scatter(band_v, [r, cs4 + (BAND_D - GOAL_D)], v)
            return r + 4

        with jax.named_scope("phase_cgoal"):
            lax.fori_loop(0, BPW * GOAL_D // LANES, goal_body, r4, unroll=4)

        with jax.named_scope("phase_out"):
            pltpu.sync_copy(band_v,
                            out_h.at[pl.ds(base, BPW), pl.ds(8, BAND_D)])
            out_t.wait()

    return k(atype, achar6, aloc, afact, agoal,
             type_emb, char_flat, loc_emb8, fact_emb8, goal_emb8)


def kernel(atype, achar, aloc, afact, agoal,
           type_emb, char_emb, loc_emb, fact_emb, goal_emb):
    def as_idx(a):
        return a.astype(jnp.int32).reshape(NW * NCHUNK, CHUNK)

    achar6 = (achar.astype(jnp.int32)[:, None] * CHAR_D
              + jnp.arange(CHAR_D, dtype=jnp.int32)[None, :]
              ).reshape(NW * NCCHUNK, CHUNK)

    def pad8(t):
        return jnp.pad(t, ((0, 0), (0, 8 - t.shape[1])))

    return _sc_encode(
        as_idx(atype), achar6, as_idx(aloc), as_idx(afact), as_idx(agoal),
        type_emb, char_emb.reshape(-1),
        pad8(loc_emb), pad8(fact_emb), pad8(goal_emb))
